# baseline (speedup 1.0000x reference)
import functools

import numpy as np
import jax
import jax.numpy as jnp
from jax import lax
from jax.experimental import pallas as pl
from jax.experimental.pallas import tpu as pltpu
from jax.experimental.pallas import tpu_sc as plsc

_HIDDEN = 128
_MAXLEN = 60
_EPS = 1e-5

_BATCH_BLOCK = 256


def _pe_const(seq_len: int) -> jnp.ndarray:
    position = np.arange(0, _MAXLEN, dtype=np.float32)[:, None]
    div_term = 1.0 / (
        10000.0 ** (np.arange(0, _HIDDEN, 2, dtype=np.float32) * 2.0 / _HIDDEN)
    )
    pe = np.zeros((_MAXLEN, _HIDDEN), dtype=np.float32)
    pe[:, 0::2] = np.sin(position * div_term)
    pe[:, 1::2] = np.cos(position * div_term)
    return jnp.asarray(pe[:seq_len])


def _sc_gather(table: jnp.ndarray, idx: jnp.ndarray) -> jnp.ndarray:
    b = idx.shape[0]
    h = table.shape[1]
    window = 128
    idx2 = idx.reshape(1, b)
    mesh = plsc.VectorSubcoreMesh(
        core_axis_name="core", subcore_axis_name="subcore"
    )

    @functools.partial(
        pl.kernel,
        out_type=jax.ShapeDtypeStruct((b, h), table.dtype),
        mesh=mesh,
    )
    def gather_kernel(tab_hbm, i_hbm, o_hbm):
        def body(i_vmem, o_vmem):
            pltpu.sync_copy(tab_hbm.at[i_vmem.at[0]], o_vmem)

        pltpu.emit_pipeline(
            body,
            grid=(b // window,),
            in_specs=[pl.BlockSpec((1, window), index_map=lambda i: (0, i))],
            out_specs=[pl.BlockSpec((window, h), index_map=lambda i: (i, 0))],
            core_axis_name=("core", "subcore"),
            dimension_semantics=(pltpu.PARALLEL,),
        )(i_hbm, o_hbm)

    return gather_kernel(table, idx2)


def _stats_body(x_ref, emb_ref, pe_ref, o_ref):
    i = pl.program_id(0)
    y = x_ref[...] + pe_ref[...][None, :, :] + emb_ref[...][:, None, :]
    s = jnp.sum(y, axis=(0, 1)).reshape(1, _HIDDEN)
    ss = jnp.sum(y * y, axis=(0, 1)).reshape(1, _HIDDEN)
    part = jnp.concatenate(
        [s, ss, jnp.zeros((6, _HIDDEN), jnp.float32)], axis=0
    )

    @pl.when(i == 0)
    def _init():
        o_ref[...] = part

    @pl.when(i != 0)
    def _acc():
        o_ref[...] = o_ref[...] + part


def _norm_body(n_total, x_ref, emb_ref, pe_ref, st_ref, w_ref, b_ref, o_ref):
    inv_n = 1.0 / n_total
    mean = st_ref[0:1, :] * inv_n
    var = st_ref[1:2, :] * inv_n - mean * mean
    scale = w_ref[...] * lax.rsqrt(var + _EPS)
    shift = b_ref[...] - mean * scale
    y = x_ref[...] + pe_ref[...][None, :, :] + emb_ref[...][:, None, :]
    o_ref[...] = y * scale[None, :, :] + shift[None, :, :]


def kernel(x, g_id, emb_table, bn_weight, bn_bias):
    b, l, h = x.shape
    bb = _BATCH_BLOCK
    nb = b // bb
    pe = _pe_const(l)
    emb = _sc_gather(emb_table, g_id)

    x_spec = pl.BlockSpec((bb, l, h), lambda i: (i, 0, 0))
    emb_spec = pl.BlockSpec((bb, h), lambda i: (i, 0))
    pe_spec = pl.BlockSpec((l, h), lambda i: (0, 0))

    stats = pl.pallas_call(
        _stats_body,
        grid=(nb,),
        in_specs=[x_spec, emb_spec, pe_spec],
        out_specs=pl.BlockSpec((8, h), lambda i: (0, 0)),
        out_shape=jax.ShapeDtypeStruct((8, h), jnp.float32),
        compiler_params=pltpu.CompilerParams(
            dimension_semantics=("arbitrary",)
        ),
    )(x, emb, pe)

    w2 = bn_weight.reshape(1, h)
    b2 = bn_bias.reshape(1, h)
    row_spec = pl.BlockSpec((1, h), lambda i: (0, 0))
    st_spec = pl.BlockSpec((8, h), lambda i: (0, 0))

    out = pl.pallas_call(
        functools.partial(_norm_body, float(b * l)),
        grid=(nb,),
        in_specs=[x_spec, emb_spec, pe_spec, st_spec, row_spec, row_spec],
        out_specs=x_spec,
        out_shape=jax.ShapeDtypeStruct((b, l, h), jnp.float32),
        compiler_params=pltpu.CompilerParams(
            dimension_semantics=("arbitrary",)
        ),
    )(x, emb, pe, stats, w2, b2)
    return out


# speedup vs baseline: 1.7459x; 1.7459x over previous
import functools

import numpy as np
import jax
import jax.numpy as jnp
from jax import lax
from jax.experimental import pallas as pl
from jax.experimental.pallas import tpu as pltpu
from jax.experimental.pallas import tpu_sc as plsc

_HIDDEN = 128
_MAXLEN = 60
_EPS = 1e-5


def _pe_const(seq_len: int) -> jnp.ndarray:
    position = np.arange(0, _MAXLEN, dtype=np.float32)[:, None]
    div_term = 1.0 / (
        10000.0 ** (np.arange(0, _HIDDEN, 2, dtype=np.float32) * 2.0 / _HIDDEN)
    )
    pe = np.zeros((_MAXLEN, _HIDDEN), dtype=np.float32)
    pe[:, 0::2] = np.sin(position * div_term)
    pe[:, 1::2] = np.cos(position * div_term)
    return jnp.asarray(pe[:seq_len])


def _sc_gather(table: jnp.ndarray, idx: jnp.ndarray) -> jnp.ndarray:
    b = idx.shape[0]
    h = table.shape[1]
    window = 128
    idx2 = idx.reshape(1, b)
    mesh = plsc.VectorSubcoreMesh(
        core_axis_name="core", subcore_axis_name="subcore"
    )

    @functools.partial(
        pl.kernel,
        out_type=jax.ShapeDtypeStruct((b, h), table.dtype),
        mesh=mesh,
    )
    def gather_kernel(tab_hbm, i_hbm, o_hbm):
        def body(i_vmem, o_vmem):
            pltpu.sync_copy(tab_hbm.at[i_vmem.at[0]], o_vmem)

        pltpu.emit_pipeline(
            body,
            grid=(b // window,),
            in_specs=[pl.BlockSpec((1, window), index_map=lambda i: (0, i))],
            out_specs=[pl.BlockSpec((window, h), index_map=lambda i: (i, 0))],
            core_axis_name=("core", "subcore"),
            dimension_semantics=(pltpu.PARALLEL,),
        )(i_hbm, o_hbm)

    return gather_kernel(table, idx2)


def _stats_body(x_ref, emb_ref, pe_ref, o_ref):
    i = pl.program_id(0)
    y = x_ref[...] + emb_ref[...] + pe_ref[0]
    s = jnp.sum(y, axis=0).reshape(1, _HIDDEN)
    ss = jnp.sum(y * y, axis=0).reshape(1, _HIDDEN)
    part = jnp.concatenate(
        [s, ss, jnp.zeros((6, _HIDDEN), jnp.float32)], axis=0
    )

    @pl.when(i == 0)
    def _init():
        o_ref[...] = part

    @pl.when(i != 0)
    def _acc():
        o_ref[...] = o_ref[...] + part


def _norm_body(n_total, x_ref, emb_ref, pe_ref, st_ref, w_ref, b_ref, o_ref):
    inv_n = 1.0 / n_total
    mean = st_ref[0:1, :] * inv_n
    var = st_ref[1:2, :] * inv_n - mean * mean
    scale = w_ref[...] * lax.rsqrt(var + _EPS)
    shift = b_ref[...] - mean * scale
    y = x_ref[...] + emb_ref[...] + pe_ref[0]
    o_ref[...] = y * scale + shift


def kernel(x, g_id, emb_table, bn_weight, bn_bias):
    b, l, h = x.shape
    pe = _pe_const(l)
    emb = _sc_gather(emb_table, g_id)

    x2 = jnp.transpose(x, (1, 0, 2)).reshape(l * b, h)

    pe3 = pe.reshape(l, 1, h)
    blk_spec = pl.BlockSpec((b, h), lambda i: (i, 0))
    emb_spec = pl.BlockSpec((b, h), lambda i: (0, 0))
    pe_spec = pl.BlockSpec((1, 1, h), lambda i: (i, 0, 0))

    stats = pl.pallas_call(
        _stats_body,
        grid=(l,),
        in_specs=[blk_spec, emb_spec, pe_spec],
        out_specs=pl.BlockSpec((8, h), lambda i: (0, 0)),
        out_shape=jax.ShapeDtypeStruct((8, h), jnp.float32),
        compiler_params=pltpu.CompilerParams(
            dimension_semantics=("arbitrary",)
        ),
    )(x2, emb, pe3)

    w2 = bn_weight.reshape(1, h)
    b2 = bn_bias.reshape(1, h)
    row_spec = pl.BlockSpec((1, h), lambda i: (0, 0))
    st_spec = pl.BlockSpec((8, h), lambda i: (0, 0))

    out2 = pl.pallas_call(
        functools.partial(_norm_body, float(b * l)),
        grid=(l,),
        in_specs=[blk_spec, emb_spec, pe_spec, st_spec, row_spec, row_spec],
        out_specs=blk_spec,
        out_shape=jax.ShapeDtypeStruct((l * b, h), jnp.float32),
        compiler_params=pltpu.CompilerParams(
            dimension_semantics=("arbitrary",)
        ),
    )(x2, emb, pe3, stats, w2, b2)

    return jnp.transpose(out2.reshape(l, b, h), (1, 0, 2))
